# baseline (speedup 1.0000x reference)
import jax
import jax.numpy as jnp
from jax.experimental import pallas as pl
from jax.experimental.pallas import tpu as pltpu

_SUBLANE = 8
_TILE_THRESHOLD = 1024


def _round_up(n, m):
    return ((n + m - 1) // m) * m


def _body3d(bn, b, in_dim):
    def body(x_ref, w_ref, o_ref):
        x = x_ref[...].reshape(bn * b, in_dim)
        o_ref[...] = jax.lax.dot_general(
            x,
            w_ref[0:in_dim, :],
            dimension_numbers=(((1,), (0,)), ((), ())),
            preferred_element_type=jnp.float32,
        ).astype(o_ref.dtype)

    return body


def _forward3d(xs, wt_pad, bn):
    n, b, in_dim = xs.shape
    in_pad, out_pad = wt_pad.shape
    rows = n * b
    grid = (n // bn,)
    return pl.pallas_call(
        _body3d(bn, b, in_dim),
        out_shape=jax.ShapeDtypeStruct((rows, out_pad), xs.dtype),
        grid=grid,
        in_specs=[
            pl.BlockSpec((bn, b, in_dim), lambda i: (i, 0, 0)),
            pl.BlockSpec((in_pad, out_pad), lambda i: (0, 0)),
        ],
        out_specs=pl.BlockSpec((bn * b, out_pad), lambda i: (i, 0)),
        compiler_params=pltpu.CompilerParams(
            dimension_semantics=("parallel",)
        ),
        cost_estimate=pl.CostEstimate(
            flops=2 * rows * in_dim * out_pad,
            transcendentals=0,
            bytes_accessed=4 * (rows * in_dim + in_pad * out_pad
                                + rows * out_pad),
        ),
    )(xs, wt_pad)


def _body2d(in_dim):
    def body(x_ref, w_ref, o_ref):
        o_ref[...] = jax.lax.dot_general(
            x_ref[...],
            w_ref[0:in_dim, :],
            dimension_numbers=(((1,), (0,)), ((), ())),
            preferred_element_type=jnp.float32,
        ).astype(o_ref.dtype)

    return body


def _forward2d(x, wt_pad, rows_pad, tile_rows):
    rows, in_dim = x.shape
    in_pad, out_pad = wt_pad.shape
    if rows_pad != rows:
        x = jnp.pad(x, ((0, rows_pad - rows), (0, 0)))
    grid = (rows_pad // tile_rows,)
    return pl.pallas_call(
        _body2d(in_dim),
        out_shape=jax.ShapeDtypeStruct((rows_pad, out_pad), x.dtype),
        grid=grid,
        in_specs=[
            pl.BlockSpec((tile_rows, in_dim), lambda i: (i, 0)),
            pl.BlockSpec((in_pad, out_pad), lambda i: (0, 0)),
        ],
        out_specs=pl.BlockSpec((tile_rows, out_pad), lambda i: (i, 0)),
        compiler_params=pltpu.CompilerParams(
            dimension_semantics=("parallel",)
        ),
    )(x, wt_pad)


@jax.jit
def kernel(xs, wt_pad):
    n, b, in_dim = xs.shape
    rows = n * b
    if rows < _TILE_THRESHOLD:
        rows_pad = _round_up(max(rows, _SUBLANE), _SUBLANE)
        return _forward2d(xs.reshape(rows, in_dim), wt_pad, rows_pad,
                          rows_pad)
    if rows % 512 == 0:
        target = max(1, 4096 // b)
        bn = 1
        for cand in range(target, 0, -1):
            if n % cand == 0:
                bn = cand
                break
        return _forward3d(xs, wt_pad, bn)
    rows_pad = _round_up(rows, 512)
    return _forward2d(xs.reshape(rows, in_dim), wt_pad, rows_pad, 512)
